# baseline (speedup 1.0000x reference)
import functools

import jax
import jax.numpy as jnp
from jax.experimental import pallas as pl
from jax.experimental.pallas import tpu as pltpu

NUM_TOKENS = 32768
HIDDEN = 768
NUM_EXPERTS = 8
TOP_K = 2
BLOCK_T = 2048


def _router_body(x_ref, w_ref, imp_ref, logits_ref, idx_ref, wts_ref,
                 var_ref, ent_ref, load_acc, ent_acc):
    step = pl.program_id(0)
    nsteps = pl.num_programs(0)

    @pl.when(step == 0)
    def _init():
        load_acc[...] = jnp.zeros_like(load_acc)
        ent_acc[...] = jnp.zeros_like(ent_acc)

    imp = imp_ref[...]
    imp_m = jnp.max(imp, axis=-1, keepdims=True)
    imp_e = jnp.exp(imp - imp_m)
    imp_w = imp_e / jnp.sum(imp_e, axis=-1, keepdims=True)
    bias = jnp.log(imp_w + 1e-8)

    x = x_ref[...]
    w = w_ref[...]
    logits = jax.lax.dot_general(
        x, w, (((1,), (1,)), ((), ())),
        preferred_element_type=jnp.float32) + bias
    logits_ref[...] = logits

    iota = jax.lax.broadcasted_iota(jnp.int32, logits.shape, 1)
    m1 = jnp.max(logits, axis=-1, keepdims=True)
    i1 = jnp.min(jnp.where(logits == m1, iota, NUM_EXPERTS),
                 axis=-1, keepdims=True)
    neg = jnp.float32(-3.0e38)
    masked = jnp.where(iota == i1, neg, logits)
    m2 = jnp.max(masked, axis=-1, keepdims=True)
    i2 = jnp.min(jnp.where(masked == m2, iota, NUM_EXPERTS),
                 axis=-1, keepdims=True)

    e2 = jnp.exp(m2 - m1)
    w1 = 1.0 / (1.0 + e2)
    w2 = 1.0 - w1

    iota2 = jax.lax.broadcasted_iota(jnp.int32, (logits.shape[0], TOP_K), 1)
    idx_ref[...] = jnp.where(iota2 == 0, i1, i2)
    wts_ref[...] = jnp.where(iota2 == 0, w1, w2)

    p = jnp.exp(logits - m1)
    p = p / jnp.sum(p, axis=-1, keepdims=True)
    load_acc[...] += jnp.sum(p, axis=0, keepdims=True)
    ent_tok = -jnp.sum(p * jnp.log(p + 1e-8), axis=-1)
    ent_acc[...] += jnp.full((1, 1), jnp.sum(ent_tok), jnp.float32)

    @pl.when(step == nsteps - 1)
    def _fin():
        load = load_acc[...] / jnp.float32(NUM_TOKENS)
        mean = jnp.sum(load) / jnp.float32(NUM_EXPERTS)
        var_ref[...] = jnp.full(
            (1, 1), jnp.sum((load - mean) ** 2) / jnp.float32(NUM_EXPERTS),
            jnp.float32)
        ent_ref[...] = ent_acc[...] / jnp.float32(NUM_TOKENS)


@functools.partial(jax.jit, static_argnames=())
def kernel(hidden_states, W, expert_importance):
    T, H = hidden_states.shape
    E = W.shape[0]
    grid = (T // BLOCK_T,)
    out_shapes = (
        jax.ShapeDtypeStruct((T, E), jnp.float32),
        jax.ShapeDtypeStruct((T, TOP_K), jnp.int32),
        jax.ShapeDtypeStruct((T, TOP_K), jnp.float32),
        jax.ShapeDtypeStruct((1, 1), jnp.float32),
        jax.ShapeDtypeStruct((1, 1), jnp.float32),
    )
    logits, idx, wts, var, ent = pl.pallas_call(
        _router_body,
        grid=grid,
        in_specs=[
            pl.BlockSpec((BLOCK_T, H), lambda i: (i, 0)),
            pl.BlockSpec((E, H), lambda i: (0, 0)),
            pl.BlockSpec((1, E), lambda i: (0, 0)),
        ],
        out_specs=(
            pl.BlockSpec((BLOCK_T, E), lambda i: (i, 0)),
            pl.BlockSpec((BLOCK_T, TOP_K), lambda i: (i, 0)),
            pl.BlockSpec((BLOCK_T, TOP_K), lambda i: (i, 0)),
            pl.BlockSpec((1, 1), lambda i: (0, 0)),
            pl.BlockSpec((1, 1), lambda i: (0, 0)),
        ),
        out_shape=out_shapes,
        scratch_shapes=[
            pltpu.VMEM((1, E), jnp.float32),
            pltpu.VMEM((1, 1), jnp.float32),
        ],
    )(hidden_states, W, expert_importance.reshape(1, E))
    return (logits, idx, wts, var[0, 0], ent[0, 0])


# speedup vs baseline: 1.0717x; 1.0717x over previous
import functools

import jax
import jax.numpy as jnp
from jax.experimental import pallas as pl
from jax.experimental.pallas import tpu as pltpu

NUM_TOKENS = 32768
HIDDEN = 768
NUM_EXPERTS = 8
TOP_K = 2
BLOCK_T = 2048


def _router_body(x_ref, w_ref, imp_ref, logits_ref, idx_ref, wts_ref,
                 var_ref, ent_ref, load_acc, ent_acc):
    step = pl.program_id(0)
    nsteps = pl.num_programs(0)

    @pl.when(step == 0)
    def _init():
        load_acc[...] = jnp.zeros_like(load_acc)
        ent_acc[...] = jnp.zeros_like(ent_acc)

    imp = imp_ref[...]
    imp_m = jnp.max(imp, axis=0, keepdims=True)
    imp_e = jnp.exp(imp - imp_m)
    imp_w = imp_e / jnp.sum(imp_e, axis=0, keepdims=True)
    bias = jnp.log(imp_w + 1e-8)

    x = x_ref[...]
    w = w_ref[...]
    logits = jax.lax.dot_general(
        x, w, (((1,), (1,)), ((), ())),
        preferred_element_type=jnp.float32)

    lt = logits.T + bias
    logits_ref[...] = lt.T

    iota = jax.lax.broadcasted_iota(jnp.int32, lt.shape, 0).astype(jnp.float32)
    m1 = jnp.max(lt, axis=0, keepdims=True)
    i1 = jnp.min(jnp.where(lt == m1, iota, jnp.float32(NUM_EXPERTS)),
                 axis=0, keepdims=True)
    neg = jnp.float32(-3.0e38)
    masked = jnp.where(iota == i1, neg, lt)
    m2 = jnp.max(masked, axis=0, keepdims=True)
    i2 = jnp.min(jnp.where(masked == m2, iota, jnp.float32(NUM_EXPERTS)),
                 axis=0, keepdims=True)

    e2 = jnp.exp(m2 - m1)
    w1 = 1.0 / (1.0 + e2)
    w2 = 1.0 - w1

    idx_t = jnp.concatenate([i1, i2], axis=0)
    wts_t = jnp.concatenate([w1, w2], axis=0)
    idx_ref[...] = idx_t.T.astype(jnp.int32)
    wts_ref[...] = wts_t.T

    p = jnp.exp(lt - m1)
    p = p / jnp.sum(p, axis=0, keepdims=True)
    load_acc[...] += jnp.sum(p, axis=1, keepdims=True)
    ent_blk = -jnp.sum(p * jnp.log(p + 1e-8))
    ent_acc[...] += jnp.full((1, 1), ent_blk, jnp.float32)

    @pl.when(step == nsteps - 1)
    def _fin():
        load = load_acc[...] / jnp.float32(NUM_TOKENS)
        mean = jnp.sum(load) / jnp.float32(NUM_EXPERTS)
        var_ref[...] = jnp.full(
            (1, 1), jnp.sum((load - mean) ** 2) / jnp.float32(NUM_EXPERTS),
            jnp.float32)
        ent_ref[...] = ent_acc[...] / jnp.float32(NUM_TOKENS)


@functools.partial(jax.jit, static_argnames=())
def kernel(hidden_states, W, expert_importance):
    T, H = hidden_states.shape
    E = W.shape[0]
    grid = (T // BLOCK_T,)
    out_shapes = (
        jax.ShapeDtypeStruct((T, E), jnp.float32),
        jax.ShapeDtypeStruct((T, TOP_K), jnp.int32),
        jax.ShapeDtypeStruct((T, TOP_K), jnp.float32),
        jax.ShapeDtypeStruct((1, 1), jnp.float32),
        jax.ShapeDtypeStruct((1, 1), jnp.float32),
    )
    logits, idx, wts, var, ent = pl.pallas_call(
        _router_body,
        grid=grid,
        in_specs=[
            pl.BlockSpec((BLOCK_T, H), lambda i: (i, 0)),
            pl.BlockSpec((E, H), lambda i: (0, 0)),
            pl.BlockSpec((E, 1), lambda i: (0, 0)),
        ],
        out_specs=(
            pl.BlockSpec((BLOCK_T, E), lambda i: (i, 0)),
            pl.BlockSpec((BLOCK_T, TOP_K), lambda i: (i, 0)),
            pl.BlockSpec((BLOCK_T, TOP_K), lambda i: (i, 0)),
            pl.BlockSpec((1, 1), lambda i: (0, 0)),
            pl.BlockSpec((1, 1), lambda i: (0, 0)),
        ),
        out_shape=out_shapes,
        scratch_shapes=[
            pltpu.VMEM((E, 1), jnp.float32),
            pltpu.VMEM((1, 1), jnp.float32),
        ],
    )(hidden_states, W, expert_importance.reshape(E, 1))
    return (logits, idx, wts, var[0, 0], ent[0, 0])
